# Initial kernel scaffold; baseline (speedup 1.0000x reference)
import jax
import jax.numpy as jnp
from jax.experimental import pallas as pl


def kernel(x, w_router, shared_gate, shared_up, shared_down, experts_gate, experts_up, experts_down):
    raise NotImplementedError("write your pallas kernel here")


# speedup vs baseline: 1.3229x; 1.3229x over previous
import functools

import jax
import jax.numpy as jnp
from jax import lax
from jax.experimental import pallas as pl
from jax.experimental.pallas import tpu as pltpu
from jax.experimental.pallas import tpu_sc as plsc

E = 8
K = 2
D = 1024
F = 2048
TOKS = 4096
A = TOKS * K
BLK = 256
G = A // BLK + E
R = G * BLK
NC, NS = 2, 16
NW = NC * NS
CH = 64

_CONTRACT_MINOR = (((1,), (1,)), ((), ()))


def _router_body(x_ref, w_ref, ti_ref, tw_ref):
    xg = x_ref[...]
    logits = lax.dot_general(xg, w_ref[...], _CONTRACT_MINOR,
                             preferred_element_type=jnp.float32)
    m = jnp.max(logits, axis=1, keepdims=True)
    p = jnp.exp(logits - m)
    probs = p / jnp.sum(p, axis=1, keepdims=True)
    iota = lax.broadcasted_iota(jnp.int32, probs.shape, 1)
    m1 = jnp.max(probs, axis=1, keepdims=True)
    i1 = jnp.min(jnp.where(probs == m1, iota, E), axis=1, keepdims=True)
    probs2 = jnp.where(iota == i1, -1.0, probs)
    m2 = jnp.max(probs2, axis=1, keepdims=True)
    i2 = jnp.min(jnp.where(probs2 == m2, iota, E), axis=1, keepdims=True)
    sw = m1 + m2
    ti_ref[:, 0:1] = i1
    ti_ref[:, 1:2] = i2
    tw_ref[:, 0:1] = m1 / sw
    tw_ref[:, 1:2] = m2 / sw


def _router(xf, w_router):
    bt = TOKS // 4
    return pl.pallas_call(
        _router_body,
        grid=(4,),
        in_specs=[pl.BlockSpec((bt, D), lambda g: (g, 0)),
                  pl.BlockSpec((E, D), lambda g: (0, 0))],
        out_specs=[pl.BlockSpec((bt, K), lambda g: (g, 0)),
                   pl.BlockSpec((bt, K), lambda g: (g, 0))],
        out_shape=[jax.ShapeDtypeStruct((TOKS, K), jnp.int32),
                   jax.ShapeDtypeStruct((TOKS, K), jnp.float32)],
    )(xf, w_router)


def _make_gather(nrows):
    per_w = nrows // NW
    nch = per_w // CH

    def call(table, idx):
        mesh = plsc.VectorSubcoreMesh(core_axis_name="c", subcore_axis_name="s",
                                      num_cores=NC, num_subcores=NS)

        @functools.partial(
            pl.kernel,
            out_type=jax.ShapeDtypeStruct((nrows, D), jnp.float32),
            mesh=mesh,
            scratch_types=[pltpu.VMEM((CH,), jnp.int32),
                           pltpu.VMEM((CH, D), jnp.float32),
                           pltpu.SemaphoreType.DMA],
        )
        def gather_k(table_hbm, idx_hbm, out_hbm, idx_v, rows_v, sem):
            wid = lax.axis_index("s") * NC + lax.axis_index("c")
            base = wid * per_w

            def body(i, carry):
                off = base + i * CH
                pltpu.sync_copy(idx_hbm.at[pl.ds(off, CH)], idx_v)
                pltpu.async_copy(table_hbm.at[idx_v], rows_v, sem).wait()
                pltpu.sync_copy(rows_v, out_hbm.at[pl.ds(off, CH)])
                return carry

            lax.fori_loop(0, nch, body, 0)

        return gather_k(table, idx)

    return call


_gather_sorted = _make_gather(R)
_gather_pairs = _make_gather(A)


def _grouped_body(eid_ref, nv_ref, x_ref, g_ref, u_ref, d_ref, w_ref, o_ref):
    g = pl.program_id(0)

    @pl.when(nv_ref[g] > 0)
    def _():
        xg = x_ref[...]
        gg = lax.dot_general(xg, g_ref[0], _CONTRACT_MINOR,
                             preferred_element_type=jnp.float32)
        uu = lax.dot_general(xg, u_ref[0], _CONTRACT_MINOR,
                             preferred_element_type=jnp.float32)
        h = gg * lax.logistic(gg) * uu
        oo = lax.dot_general(h, d_ref[0], _CONTRACT_MINOR,
                             preferred_element_type=jnp.float32)
        o_ref[...] = oo * w_ref[...]


def _grouped(x_sorted, gate_w, up_w, down_w, row_w, eid, nvalid):
    grid_spec = pltpu.PrefetchScalarGridSpec(
        num_scalar_prefetch=2,
        grid=(G,),
        in_specs=[
            pl.BlockSpec((BLK, D), lambda g, e, nv: (g, 0)),
            pl.BlockSpec((1, F, D), lambda g, e, nv: (e[g], 0, 0)),
            pl.BlockSpec((1, F, D), lambda g, e, nv: (e[g], 0, 0)),
            pl.BlockSpec((1, D, F), lambda g, e, nv: (e[g], 0, 0)),
            pl.BlockSpec((BLK, 1), lambda g, e, nv: (g, 0)),
        ],
        out_specs=pl.BlockSpec((BLK, D), lambda g, e, nv: (g, 0)),
    )
    return pl.pallas_call(
        _grouped_body,
        grid_spec=grid_spec,
        out_shape=jax.ShapeDtypeStruct((R, D), jnp.float32),
    )(eid, nvalid, x_sorted, gate_w, up_w, down_w, row_w)


def _shared_body(x_ref, g_ref, u_ref, d_ref, p_ref, o_ref):
    xg = x_ref[...]
    gg = lax.dot_general(xg, g_ref[...], _CONTRACT_MINOR,
                         preferred_element_type=jnp.float32)
    uu = lax.dot_general(xg, u_ref[...], _CONTRACT_MINOR,
                         preferred_element_type=jnp.float32)
    h = gg * lax.logistic(gg) * uu
    sh = lax.dot_general(h, d_ref[...], _CONTRACT_MINOR,
                         preferred_element_type=jnp.float32)
    o_ref[...] = sh + p_ref[:, 0, :] + p_ref[:, 1, :]


def _shared_combine(xf, gate_w, up_w, down_w, pairs):
    bt = 512
    return pl.pallas_call(
        _shared_body,
        grid=(TOKS // bt,),
        in_specs=[pl.BlockSpec((bt, D), lambda g: (g, 0)),
                  pl.BlockSpec((F, D), lambda g: (0, 0)),
                  pl.BlockSpec((F, D), lambda g: (0, 0)),
                  pl.BlockSpec((D, F), lambda g: (0, 0)),
                  pl.BlockSpec((bt, K, D), lambda g: (g, 0, 0))],
        out_specs=pl.BlockSpec((bt, D), lambda g: (g, 0)),
        out_shape=jax.ShapeDtypeStruct((TOKS, D), jnp.float32),
    )(xf, gate_w, up_w, down_w, pairs)


def kernel(x, w_router, shared_gate, shared_up, shared_down,
           experts_gate, experts_up, experts_down):
    b, s, d = x.shape
    xf = x.reshape(-1, d)

    top_i, top_w = _router(xf, w_router)

    ae = top_i.reshape(A)
    aw = top_w.reshape(A)
    onehot = (ae[:, None] == jnp.arange(E, dtype=jnp.int32)[None, :])
    ranks = jnp.cumsum(onehot.astype(jnp.int32), axis=0)
    counts = ranks[-1]
    rank = jnp.take_along_axis(ranks, ae[:, None], axis=1)[:, 0] - 1
    padded = ((counts + BLK - 1) // BLK) * BLK
    ends = jnp.cumsum(padded).astype(jnp.int32)
    offs = ends - padded
    dest = (offs[ae] + rank).astype(jnp.int32)

    tok = jnp.zeros((R,), jnp.int32).at[dest].set(
        jnp.arange(A, dtype=jnp.int32) // K)
    rw = jnp.zeros((R,), jnp.float32).at[dest].set(aw)

    gstart = jnp.arange(G, dtype=jnp.int32) * BLK
    eid = jnp.searchsorted(ends, gstart, side="right").astype(jnp.int32)
    eid_c = jnp.minimum(eid, E - 1)
    nvalid = jnp.where(
        eid < E,
        jnp.clip(offs[eid_c] + counts[eid_c] - gstart, 0, BLK),
        0).astype(jnp.int32)

    x_sorted = _gather_sorted(xf, tok)
    out_sorted = _grouped(x_sorted, experts_gate, experts_up, experts_down,
                          rw.reshape(R, 1), eid_c, nvalid)
    pairs = _gather_pairs(out_sorted, dest)
    out = _shared_combine(xf, shared_gate, shared_up, shared_down,
                          pairs.reshape(TOKS, K, D))
    return out.reshape(b, s, d)
